# baseline (device time: 1185547 ns/iter reference)
import jax
import jax.numpy as jnp
from jax import lax
from jax.experimental import pallas as pl
from jax.experimental.pallas import tpu as pltpu

_CHUNK_ROWS = 1024


def kernel(x):
    m, n = x.shape
    n_half = n // 2
    nc = m // _CHUNK_ROWS
    r = _CHUNK_ROWS

    my_x = lax.axis_index("x")
    other = 1 - my_x

    xbf = x.astype(jnp.bfloat16)
    send = lax.dynamic_slice(xbf, (0, other * n_half), (m, n_half))
    keep = lax.dynamic_slice(xbf, (0, my_x * n_half), (m, n_half))

    def body(send_ref, keep_ref, out_ref, stage, local_sem, load_sems,
             send_sems, recv_sems):
        my_x = lax.axis_index("x")
        my_y = lax.axis_index("y")
        my_z = lax.axis_index("z")
        other = 1 - my_x

        barrier_sem = pltpu.get_barrier_semaphore()
        pl.semaphore_signal(
            barrier_sem, inc=1,
            device_id=(other, my_y, my_z),
            device_id_type=pl.DeviceIdType.MESH,
        )
        pl.semaphore_wait(barrier_sem, 1)

        local = pltpu.make_async_copy(
            keep_ref,
            out_ref.at[pl.ds(my_x * m, m), :],
            local_sem,
        )
        local.start()

        rdmas = []
        for c in range(nc):
            slot = c % 2
            if c >= 2:
                rdmas[c - 2].wait_send()
            load = pltpu.make_async_copy(
                send_ref.at[pl.ds(c * r, r), :], stage.at[slot], load_sems.at[slot]
            )
            load.start()
            load.wait()
            rdma = pltpu.make_async_remote_copy(
                src_ref=stage.at[slot],
                dst_ref=out_ref.at[pl.ds(my_x * m + c * r, r), :],
                send_sem=send_sems.at[c],
                recv_sem=recv_sems.at[c],
                device_id=(other, my_y, my_z),
                device_id_type=pl.DeviceIdType.MESH,
            )
            rdma.start()
            rdmas.append(rdma)

        rdmas[nc - 2].wait_send()
        rdmas[nc - 1].wait_send()
        for c in range(nc):
            rdmas[c].wait_recv()
        local.wait()

    return pl.pallas_call(
        body,
        out_shape=jax.ShapeDtypeStruct((2 * m, n_half), jnp.bfloat16),
        in_specs=[
            pl.BlockSpec(memory_space=pltpu.MemorySpace.HBM),
            pl.BlockSpec(memory_space=pltpu.MemorySpace.HBM),
        ],
        out_specs=pl.BlockSpec(memory_space=pltpu.MemorySpace.HBM),
        scratch_shapes=[
            pltpu.VMEM((2, r, n_half), jnp.bfloat16),
            pltpu.SemaphoreType.DMA,
            pltpu.SemaphoreType.DMA((2,)),
            pltpu.SemaphoreType.DMA((nc,)),
            pltpu.SemaphoreType.DMA((nc,)),
        ],
        compiler_params=pltpu.CompilerParams(collective_id=0),
    )(send, keep)
